# Initial kernel scaffold; baseline (speedup 1.0000x reference)
import jax
import jax.numpy as jnp
from jax.experimental import pallas as pl


def kernel(x, edge_index, W1l, W1r, b1, g1, be1, W2l, W2r, b2, g2, be2, W3l, W3r, b3, Wfc, bfc):
    raise NotImplementedError("write your pallas kernel here")


# speedup vs baseline: 5.6962x; 5.6962x over previous
import functools

import jax
import jax.numpy as jnp
from jax import lax
from jax.experimental import pallas as pl
from jax.experimental.pallas import tpu as pltpu
from jax.experimental.pallas import tpu_sc as plsc

_N = 100000
_E = 3200000
_NC = 2
_NS = 16
_L = 16
_B = 128
_EP = ((_E + _NC * _NS * _B - 1) // (_NC * _NS * _B)) * (_NC * _NS * _B)
_PW = _EP // (_NC * _NS)
_NB = _PW // _B
_NPAD = 100096
_RPS = _NPAD // _NS
_ZB = _RPS // 8
_EPS = 1e-5

_BLK = 1000
_G = _N // _BLK


def _segsum(tables, srcp, dstp, zeros):
    C = tables.shape[0]
    mesh = plsc.VectorSubcoreMesh(core_axis_name="c", subcore_axis_name="s")

    @functools.partial(
        pl.kernel,
        out_type=jax.ShapeDtypeStruct((_NC, _NPAD, _L * C), jnp.float32),
        mesh=mesh,
        scratch_types=[
            pltpu.VMEM((_B,), jnp.int32),
            pltpu.VMEM((_B,), jnp.int32),
            pltpu.VMEM((_B, _L), jnp.float32),
            pltpu.VMEM((_ZB, _L), jnp.float32),
            pltpu.VMEM_SHARED((_NPAD, _L), jnp.float32),
            pltpu.SemaphoreType.DMA,
        ],
        compiler_params=pltpu.CompilerParams(use_tc_tiling_on_sc=False),
    )
    def k(tab_hbm, src_hbm, dst_hbm, zero_hbm, out_hbm,
          src_v, dst_v, rows_v, zbuf, acc, sem):
        ci = lax.axis_index("c")
        si = lax.axis_index("s")
        wid = ci * _NS + si
        base = wid * _PW
        pltpu.sync_copy(zero_hbm, zbuf)

        for c in range(C):
            @pl.loop(0, _RPS // _ZB)
            def _(j):
                pltpu.sync_copy(zbuf, acc.at[pl.ds(si * _RPS + j * _ZB, _ZB)])

            plsc.subcore_barrier()

            @pl.loop(0, _NB)
            def _(it):
                off = base + it * _B
                pltpu.sync_copy(src_hbm.at[pl.ds(off, _B)], src_v)
                pltpu.sync_copy(dst_hbm.at[pl.ds(off, _B)], dst_v)
                pltpu.async_copy(tab_hbm.at[c].at[src_v], rows_v, sem).wait()
                pltpu.sync_copy(rows_v, acc.at[dst_v], add=True)

            plsc.subcore_barrier()
            pltpu.sync_copy(
                acc.at[pl.ds(si * _RPS, _RPS)],
                out_hbm.at[ci].at[pl.ds(si * _RPS, _RPS), pl.ds(c * _L, _L)])
            plsc.subcore_barrier()

    return k(tables, srcp, dstp, zeros)


def _lin1(p1, x, W1l, W1r, b1):
    def body(p_ref, x_ref, wl_ref, wr_ref, b_ref, y_ref, st_ref):
        p = p_ref[0] + p_ref[1]
        cnt = p[:, 8:9]
        inv = 1.0 / jnp.maximum(cnt, 1.0)
        mean8 = p[:, :8] * inv
        y = (lax.dot_general(mean8, wl_ref[...], (((1,), (1,)), ((), ())),
                             preferred_element_type=jnp.float32)
             + lax.dot_general(x_ref[...], wr_ref[...], (((1,), (1,)), ((), ())),
                               preferred_element_type=jnp.float32)
             + b_ref[...])
        y_ref[...] = y

        @pl.when(pl.program_id(0) == 0)
        def _():
            st_ref[...] = jnp.zeros_like(st_ref)

        st_ref[...] += jnp.stack([jnp.sum(y, axis=0), jnp.sum(y * y, axis=0)])

    return pl.pallas_call(
        body,
        grid=(_G,),
        in_specs=[
            pl.BlockSpec((2, _BLK, _L), lambda i: (0, i, 0)),
            pl.BlockSpec((_BLK, 8), lambda i: (i, 0)),
            pl.BlockSpec((64, 8), lambda i: (0, 0)),
            pl.BlockSpec((64, 8), lambda i: (0, 0)),
            pl.BlockSpec((64,), lambda i: (0,)),
        ],
        out_specs=[
            pl.BlockSpec((_BLK, 64), lambda i: (i, 0)),
            pl.BlockSpec((2, 64), lambda i: (0, 0)),
        ],
        out_shape=[
            jax.ShapeDtypeStruct((_N, 64), jnp.float32),
            jax.ShapeDtypeStruct((2, 64), jnp.float32),
        ],
    )(p1, x, W1l, W1r, b1)


def _bnrelu(y, st, g, be):
    def body(y_ref, st_ref, g_ref, be_ref, h_ref, hc_ref):
        yv = y_ref[...]
        mu = st_ref[0:1, :] / _N
        var = st_ref[1:2, :] / _N - mu * mu
        h = jnp.maximum((yv - mu) * lax.rsqrt(var + _EPS) * g_ref[...]
                        + be_ref[...], 0.0)
        h_ref[...] = h
        for c in range(4):
            hc_ref[c] = h[:, _L * c:_L * (c + 1)]

    return pl.pallas_call(
        body,
        grid=(_G,),
        in_specs=[
            pl.BlockSpec((_BLK, 64), lambda i: (i, 0)),
            pl.BlockSpec((2, 64), lambda i: (0, 0)),
            pl.BlockSpec((64,), lambda i: (0,)),
            pl.BlockSpec((64,), lambda i: (0,)),
        ],
        out_specs=[
            pl.BlockSpec((_BLK, 64), lambda i: (i, 0)),
            pl.BlockSpec((4, _BLK, _L), lambda i: (0, i, 0)),
        ],
        out_shape=[
            jax.ShapeDtypeStruct((_N, 64), jnp.float32),
            jax.ShapeDtypeStruct((4, _N, _L), jnp.float32),
        ],
    )(y, st, g, be)


def _lin2(p2, p1, h1, W2l, W2r, b2):
    def body(p2_ref, p1_ref, h_ref, wl_ref, wr_ref, b_ref, y_ref, st_ref):
        cnt = p1_ref[0, :, 8:9] + p1_ref[1, :, 8:9]
        inv = 1.0 / jnp.maximum(cnt, 1.0)
        agg = (p2_ref[0] + p2_ref[1]) * inv
        y = (lax.dot_general(agg, wl_ref[...], (((1,), (1,)), ((), ())),
                             preferred_element_type=jnp.float32)
             + lax.dot_general(h_ref[...], wr_ref[...], (((1,), (1,)), ((), ())),
                               preferred_element_type=jnp.float32)
             + b_ref[...])
        y_ref[...] = y

        @pl.when(pl.program_id(0) == 0)
        def _():
            st_ref[...] = jnp.zeros_like(st_ref)

        st_ref[...] += jnp.stack([jnp.sum(y, axis=0), jnp.sum(y * y, axis=0)])

    return pl.pallas_call(
        body,
        grid=(_G,),
        in_specs=[
            pl.BlockSpec((2, _BLK, 64), lambda i: (0, i, 0)),
            pl.BlockSpec((2, _BLK, _L), lambda i: (0, i, 0)),
            pl.BlockSpec((_BLK, 64), lambda i: (i, 0)),
            pl.BlockSpec((64, 64), lambda i: (0, 0)),
            pl.BlockSpec((64, 64), lambda i: (0, 0)),
            pl.BlockSpec((64,), lambda i: (0,)),
        ],
        out_specs=[
            pl.BlockSpec((_BLK, 64), lambda i: (i, 0)),
            pl.BlockSpec((2, 64), lambda i: (0, 0)),
        ],
        out_shape=[
            jax.ShapeDtypeStruct((_N, 64), jnp.float32),
            jax.ShapeDtypeStruct((2, 64), jnp.float32),
        ],
    )(p2, p1, h1, W2l, W2r, b2)


def _bnrelu_t3(y2, st2, g2, be2, W3l, W3r):
    def body(y_ref, st_ref, g_ref, be_ref, wl_ref, wr_ref, t_ref, r_ref):
        yv = y_ref[...]
        mu = st_ref[0:1, :] / _N
        var = st_ref[1:2, :] / _N - mu * mu
        h = jnp.maximum((yv - mu) * lax.rsqrt(var + _EPS) * g_ref[...]
                        + be_ref[...], 0.0)
        t3 = lax.dot_general(h, wl_ref[...], (((1,), (1,)), ((), ())),
                             preferred_element_type=jnp.float32)
        r3 = lax.dot_general(h, wr_ref[...], (((1,), (1,)), ((), ())),
                             preferred_element_type=jnp.float32)
        t_ref[0] = t3[:, :16]
        t_ref[1] = t3[:, 16:32]
        r_ref[...] = r3

    return pl.pallas_call(
        body,
        grid=(_G,),
        in_specs=[
            pl.BlockSpec((_BLK, 64), lambda i: (i, 0)),
            pl.BlockSpec((2, 64), lambda i: (0, 0)),
            pl.BlockSpec((64,), lambda i: (0,)),
            pl.BlockSpec((64,), lambda i: (0,)),
            pl.BlockSpec((32, 64), lambda i: (0, 0)),
            pl.BlockSpec((32, 64), lambda i: (0, 0)),
        ],
        out_specs=[
            pl.BlockSpec((2, _BLK, _L), lambda i: (0, i, 0)),
            pl.BlockSpec((_BLK, 32), lambda i: (i, 0)),
        ],
        out_shape=[
            jax.ShapeDtypeStruct((2, _N, _L), jnp.float32),
            jax.ShapeDtypeStruct((_N, 32), jnp.float32),
        ],
    )(y2, st2, g2, be2, W3l, W3r)


def _final(p3, p1, r3, b3, Wfc, bfc):
    def body(p3_ref, p1_ref, r_ref, b_ref, w_ref, bf_ref, o_ref):
        cnt = p1_ref[0, :, 8:9] + p1_ref[1, :, 8:9]
        inv = 1.0 / jnp.maximum(cnt, 1.0)
        agg = (p3_ref[0] + p3_ref[1]) * inv
        h3 = jnp.maximum(agg + r_ref[...] + b_ref[...], 0.0)
        logit = jnp.sum(h3 * w_ref[...], axis=1, keepdims=True) + bf_ref[0]
        o_ref[...] = 1.0 / (1.0 + jnp.exp(-logit))

    return pl.pallas_call(
        body,
        grid=(_G,),
        in_specs=[
            pl.BlockSpec((2, _BLK, 32), lambda i: (0, i, 0)),
            pl.BlockSpec((2, _BLK, _L), lambda i: (0, i, 0)),
            pl.BlockSpec((_BLK, 32), lambda i: (i, 0)),
            pl.BlockSpec((32,), lambda i: (0,)),
            pl.BlockSpec((1, 32), lambda i: (0, 0)),
            pl.BlockSpec((1,), lambda i: (0,)),
        ],
        out_specs=pl.BlockSpec((_BLK, 1), lambda i: (i, 0)),
        out_shape=jax.ShapeDtypeStruct((_N, 1), jnp.float32),
    )(p3, p1, r3, b3, Wfc, bfc)


def kernel(x, edge_index, W1l, W1r, b1, g1, be1, W2l, W2r, b2, g2, be2,
           W3l, W3r, b3, Wfc, bfc):
    src = edge_index[0]
    dst = edge_index[1]
    pad = _EP - _E
    srcp = jnp.concatenate([src, jnp.zeros((pad,), jnp.int32)])
    dstp = jnp.concatenate([dst, jnp.full((pad,), _N, jnp.int32)])
    zeros = jnp.zeros((_ZB, _L), jnp.float32)
    xaug = jnp.concatenate(
        [x, jnp.ones((_N, 1), jnp.float32), jnp.zeros((_N, 7), jnp.float32)],
        axis=1)[None]

    p1 = _segsum(xaug, srcp, dstp, zeros)
    y1, st1 = _lin1(p1, x, W1l, W1r, b1)
    h1, h1c = _bnrelu(y1, st1, g1, be1)
    p2 = _segsum(h1c, srcp, dstp, zeros)
    y2, st2 = _lin2(p2, p1, h1, W2l, W2r, b2)
    t3c, r3 = _bnrelu_t3(y2, st2, g2, be2, W3l, W3r)
    p3 = _segsum(t3c, srcp, dstp, zeros)
    o = _final(p3, p1, r3, b3, Wfc, bfc)
    return jnp.squeeze(o, axis=-1)


# speedup vs baseline: 11.7288x; 2.0591x over previous
import functools

import jax
import jax.numpy as jnp
from jax import lax
from jax.experimental import pallas as pl
from jax.experimental.pallas import tpu as pltpu
from jax.experimental.pallas import tpu_sc as plsc

_N = 100000
_E = 3200000
_NC = 2
_NS = 16
_L = 16
_B = 128
_CH = 16
_PW = 102400
_EP = _PW * _NC * _NS
_NBATCH = _PW // _B
_NCHUNK = _NBATCH // _CH
_RING = 4
_LOOK = 3
_NPAD = 100096
_RPS = _NPAD // _NS
_ZB = _RPS // 8
_EPS = 1e-5

_BLK = 1000
_G = _N // _BLK


def _segsum(tables, srcp2d, dstp2d, zeros):
    C = tables.shape[0]
    mesh = plsc.VectorSubcoreMesh(core_axis_name="c", subcore_axis_name="s")

    @functools.partial(
        pl.kernel,
        out_type=jax.ShapeDtypeStruct((_NC, _NPAD, _L * C), jnp.float32),
        mesh=mesh,
        scratch_types=[
            pltpu.VMEM((2, _CH, _B), jnp.int32),
            pltpu.VMEM((2, _CH, _B), jnp.int32),
            pltpu.VMEM((_RING, _B, _L), jnp.float32),
            pltpu.VMEM((_ZB, _L), jnp.float32),
            pltpu.VMEM_SHARED((_NPAD, _L), jnp.float32),
            pltpu.SemaphoreType.DMA((2,)),
            pltpu.SemaphoreType.DMA((_RING,)),
            pltpu.SemaphoreType.DMA((_RING,)),
        ],
        compiler_params=pltpu.CompilerParams(use_tc_tiling_on_sc=False),
    )
    def k(tab_hbm, src_hbm, dst_hbm, zero_hbm, out_hbm,
          sbufs, dbufs, rows, zbuf, acc, isem, gsem, ssem):
        ci = lax.axis_index("c")
        si = lax.axis_index("s")
        wid = ci * _NS + si
        rbase = wid * _NBATCH
        pltpu.sync_copy(zero_hbm, zbuf)

        def idx_copies(chunk, p):
            s_ = pltpu.make_async_copy(
                src_hbm.at[pl.ds(rbase + chunk * _CH, _CH)], sbufs.at[p],
                isem.at[p])
            d_ = pltpu.make_async_copy(
                dst_hbm.at[pl.ds(rbase + chunk * _CH, _CH)], dbufs.at[p],
                isem.at[p])
            return s_, d_

        for c in range(C):
            tab_c = tab_hbm.at[c]

            @pl.loop(0, _RPS // _ZB)
            def _(j):
                pltpu.sync_copy(zbuf, acc.at[pl.ds(si * _RPS + j * _ZB, _ZB)])

            plsc.subcore_barrier()

            s0, d0 = idx_copies(0, 0)
            s0.start()
            d0.start()

            @pl.loop(0, _NCHUNK, step=2)
            def _(kk):
                for pp in range(2):
                    chunk = kk + pp

                    @pl.when(chunk + 1 < _NCHUNK)
                    def _():
                        s_, d_ = idx_copies(chunk + 1, 1 - pp)
                        s_.start()
                        d_.start()

                    s_w, d_w = idx_copies(chunk, pp)
                    s_w.wait()
                    d_w.wait()
                    sbuf = sbufs.at[pp]
                    dbuf = dbufs.at[pp]

                    gd, sd = {}, {}
                    for j in range(_LOOK):
                        gd[j] = pltpu.async_copy(
                            tab_c.at[sbuf.at[j]], rows.at[j % _RING],
                            gsem.at[j % _RING])
                    for j in range(_CH):
                        b = j % _RING
                        gd[j].wait()
                        sd[j] = pltpu.async_copy(
                            rows.at[b], acc.at[dbuf.at[j]], ssem.at[b],
                            add=True)
                        jj = j + _LOOK
                        if jj < _CH:
                            if jj - _RING >= 0:
                                sd[jj - _RING].wait()
                            gd[jj] = pltpu.async_copy(
                                tab_c.at[sbuf.at[jj]], rows.at[jj % _RING],
                                gsem.at[jj % _RING])
                    for j in range(max(0, _CH - _RING), _CH):
                        sd[j].wait()

            plsc.subcore_barrier()
            pltpu.sync_copy(
                acc.at[pl.ds(si * _RPS, _RPS)],
                out_hbm.at[ci].at[pl.ds(si * _RPS, _RPS), pl.ds(c * _L, _L)])
            plsc.subcore_barrier()

    return k(tables, srcp2d, dstp2d, zeros)


def _lin1(p1, x, W1l, W1r, b1):
    def body(p_ref, x_ref, wl_ref, wr_ref, b_ref, y_ref, st_ref):
        p = p_ref[0] + p_ref[1]
        cnt = p[:, 8:9]
        inv = 1.0 / jnp.maximum(cnt, 1.0)
        mean8 = p[:, :8] * inv
        y = (lax.dot_general(mean8, wl_ref[...], (((1,), (1,)), ((), ())),
                             preferred_element_type=jnp.float32)
             + lax.dot_general(x_ref[...], wr_ref[...], (((1,), (1,)), ((), ())),
                               preferred_element_type=jnp.float32)
             + b_ref[...])
        y_ref[...] = y

        @pl.when(pl.program_id(0) == 0)
        def _():
            st_ref[...] = jnp.zeros_like(st_ref)

        st_ref[...] += jnp.stack([jnp.sum(y, axis=0), jnp.sum(y * y, axis=0)])

    return pl.pallas_call(
        body,
        grid=(_G,),
        in_specs=[
            pl.BlockSpec((2, _BLK, _L), lambda i: (0, i, 0)),
            pl.BlockSpec((_BLK, 8), lambda i: (i, 0)),
            pl.BlockSpec((64, 8), lambda i: (0, 0)),
            pl.BlockSpec((64, 8), lambda i: (0, 0)),
            pl.BlockSpec((64,), lambda i: (0,)),
        ],
        out_specs=[
            pl.BlockSpec((_BLK, 64), lambda i: (i, 0)),
            pl.BlockSpec((2, 64), lambda i: (0, 0)),
        ],
        out_shape=[
            jax.ShapeDtypeStruct((_N, 64), jnp.float32),
            jax.ShapeDtypeStruct((2, 64), jnp.float32),
        ],
    )(p1, x, W1l, W1r, b1)


def _bnrelu(y, st, g, be):
    def body(y_ref, st_ref, g_ref, be_ref, h_ref, hc_ref):
        yv = y_ref[...]
        mu = st_ref[0:1, :] / _N
        var = st_ref[1:2, :] / _N - mu * mu
        h = jnp.maximum((yv - mu) * lax.rsqrt(var + _EPS) * g_ref[...]
                        + be_ref[...], 0.0)
        h_ref[...] = h
        for c in range(4):
            hc_ref[c] = h[:, _L * c:_L * (c + 1)]

    return pl.pallas_call(
        body,
        grid=(_G,),
        in_specs=[
            pl.BlockSpec((_BLK, 64), lambda i: (i, 0)),
            pl.BlockSpec((2, 64), lambda i: (0, 0)),
            pl.BlockSpec((64,), lambda i: (0,)),
            pl.BlockSpec((64,), lambda i: (0,)),
        ],
        out_specs=[
            pl.BlockSpec((_BLK, 64), lambda i: (i, 0)),
            pl.BlockSpec((4, _BLK, _L), lambda i: (0, i, 0)),
        ],
        out_shape=[
            jax.ShapeDtypeStruct((_N, 64), jnp.float32),
            jax.ShapeDtypeStruct((4, _N, _L), jnp.float32),
        ],
    )(y, st, g, be)


def _lin2(p2, p1, h1, W2l, W2r, b2):
    def body(p2_ref, p1_ref, h_ref, wl_ref, wr_ref, b_ref, y_ref, st_ref):
        cnt = p1_ref[0, :, 8:9] + p1_ref[1, :, 8:9]
        inv = 1.0 / jnp.maximum(cnt, 1.0)
        agg = (p2_ref[0] + p2_ref[1]) * inv
        y = (lax.dot_general(agg, wl_ref[...], (((1,), (1,)), ((), ())),
                             preferred_element_type=jnp.float32)
             + lax.dot_general(h_ref[...], wr_ref[...], (((1,), (1,)), ((), ())),
                               preferred_element_type=jnp.float32)
             + b_ref[...])
        y_ref[...] = y

        @pl.when(pl.program_id(0) == 0)
        def _():
            st_ref[...] = jnp.zeros_like(st_ref)

        st_ref[...] += jnp.stack([jnp.sum(y, axis=0), jnp.sum(y * y, axis=0)])

    return pl.pallas_call(
        body,
        grid=(_G,),
        in_specs=[
            pl.BlockSpec((2, _BLK, 64), lambda i: (0, i, 0)),
            pl.BlockSpec((2, _BLK, _L), lambda i: (0, i, 0)),
            pl.BlockSpec((_BLK, 64), lambda i: (i, 0)),
            pl.BlockSpec((64, 64), lambda i: (0, 0)),
            pl.BlockSpec((64, 64), lambda i: (0, 0)),
            pl.BlockSpec((64,), lambda i: (0,)),
        ],
        out_specs=[
            pl.BlockSpec((_BLK, 64), lambda i: (i, 0)),
            pl.BlockSpec((2, 64), lambda i: (0, 0)),
        ],
        out_shape=[
            jax.ShapeDtypeStruct((_N, 64), jnp.float32),
            jax.ShapeDtypeStruct((2, 64), jnp.float32),
        ],
    )(p2, p1, h1, W2l, W2r, b2)


def _bnrelu_t3(y2, st2, g2, be2, W3l, W3r):
    def body(y_ref, st_ref, g_ref, be_ref, wl_ref, wr_ref, t_ref, r_ref):
        yv = y_ref[...]
        mu = st_ref[0:1, :] / _N
        var = st_ref[1:2, :] / _N - mu * mu
        h = jnp.maximum((yv - mu) * lax.rsqrt(var + _EPS) * g_ref[...]
                        + be_ref[...], 0.0)
        t3 = lax.dot_general(h, wl_ref[...], (((1,), (1,)), ((), ())),
                             preferred_element_type=jnp.float32)
        r3 = lax.dot_general(h, wr_ref[...], (((1,), (1,)), ((), ())),
                             preferred_element_type=jnp.float32)
        t_ref[0] = t3[:, :16]
        t_ref[1] = t3[:, 16:32]
        r_ref[...] = r3

    return pl.pallas_call(
        body,
        grid=(_G,),
        in_specs=[
            pl.BlockSpec((_BLK, 64), lambda i: (i, 0)),
            pl.BlockSpec((2, 64), lambda i: (0, 0)),
            pl.BlockSpec((64,), lambda i: (0,)),
            pl.BlockSpec((64,), lambda i: (0,)),
            pl.BlockSpec((32, 64), lambda i: (0, 0)),
            pl.BlockSpec((32, 64), lambda i: (0, 0)),
        ],
        out_specs=[
            pl.BlockSpec((2, _BLK, _L), lambda i: (0, i, 0)),
            pl.BlockSpec((_BLK, 32), lambda i: (i, 0)),
        ],
        out_shape=[
            jax.ShapeDtypeStruct((2, _N, _L), jnp.float32),
            jax.ShapeDtypeStruct((_N, 32), jnp.float32),
        ],
    )(y2, st2, g2, be2, W3l, W3r)


def _final(p3, p1, r3, b3, Wfc, bfc):
    def body(p3_ref, p1_ref, r_ref, b_ref, w_ref, bf_ref, o_ref):
        cnt = p1_ref[0, :, 8:9] + p1_ref[1, :, 8:9]
        inv = 1.0 / jnp.maximum(cnt, 1.0)
        agg = (p3_ref[0] + p3_ref[1]) * inv
        h3 = jnp.maximum(agg + r_ref[...] + b_ref[...], 0.0)
        logit = jnp.sum(h3 * w_ref[...], axis=1, keepdims=True) + bf_ref[0]
        o_ref[...] = 1.0 / (1.0 + jnp.exp(-logit))

    return pl.pallas_call(
        body,
        grid=(_G,),
        in_specs=[
            pl.BlockSpec((2, _BLK, 32), lambda i: (0, i, 0)),
            pl.BlockSpec((2, _BLK, _L), lambda i: (0, i, 0)),
            pl.BlockSpec((_BLK, 32), lambda i: (i, 0)),
            pl.BlockSpec((32,), lambda i: (0,)),
            pl.BlockSpec((1, 32), lambda i: (0, 0)),
            pl.BlockSpec((1,), lambda i: (0,)),
        ],
        out_specs=pl.BlockSpec((_BLK, 1), lambda i: (i, 0)),
        out_shape=jax.ShapeDtypeStruct((_N, 1), jnp.float32),
    )(p3, p1, r3, b3, Wfc, bfc)


def kernel(x, edge_index, W1l, W1r, b1, g1, be1, W2l, W2r, b2, g2, be2,
           W3l, W3r, b3, Wfc, bfc):
    src = edge_index[0]
    dst = edge_index[1]
    pad = _EP - _E
    srcp = jnp.concatenate([src, jnp.zeros((pad,), jnp.int32)]).reshape(-1, _B)
    dstp = jnp.concatenate([dst, jnp.full((pad,), _N, jnp.int32)]).reshape(-1, _B)
    zeros = jnp.zeros((_ZB, _L), jnp.float32)
    xaug = jnp.concatenate(
        [x, jnp.ones((_N, 1), jnp.float32), jnp.zeros((_N, 7), jnp.float32)],
        axis=1)[None]

    p1 = _segsum(xaug, srcp, dstp, zeros)
    y1, st1 = _lin1(p1, x, W1l, W1r, b1)
    h1, h1c = _bnrelu(y1, st1, g1, be1)
    p2 = _segsum(h1c, srcp, dstp, zeros)
    y2, st2 = _lin2(p2, p1, h1, W2l, W2r, b2)
    t3c, r3 = _bnrelu_t3(y2, st2, g2, be2, W3l, W3r)
    p3 = _segsum(t3c, srcp, dstp, zeros)
    o = _final(p3, p1, r3, b3, Wfc, bfc)
    return jnp.squeeze(o, axis=-1)


# speedup vs baseline: 12.0419x; 1.0267x over previous
import functools

import jax
import jax.numpy as jnp
from jax import lax
from jax.experimental import pallas as pl
from jax.experimental.pallas import tpu as pltpu
from jax.experimental.pallas import tpu_sc as plsc

_N = 100000
_E = 3200000
_NC = 2
_NS = 16
_L = 16
_B = 128
_CH = 16
_PW = 102400
_EP = _PW * _NC * _NS
_NBATCH = _PW // _B
_NCHUNK = _NBATCH // _CH
_RING = 5
_LOOK = 4
_NPAD = 100096
_RPS = _NPAD // _NS
_ZB = _RPS // 8
_EPS = 1e-5

_BLK = 1000
_G = _N // _BLK


def _segsum(tables, srcp2d, dstp2d, zeros):
    C = tables.shape[0]
    mesh = plsc.VectorSubcoreMesh(core_axis_name="c", subcore_axis_name="s")

    @functools.partial(
        pl.kernel,
        out_type=jax.ShapeDtypeStruct((_NC, _NPAD, _L * C), jnp.float32),
        mesh=mesh,
        scratch_types=[
            pltpu.VMEM((2, _CH, _B), jnp.int32),
            pltpu.VMEM((2, _CH, _B), jnp.int32),
            pltpu.VMEM((_RING, _B, _L), jnp.float32),
            pltpu.VMEM((_ZB, _L), jnp.float32),
            pltpu.VMEM_SHARED((_NPAD, _L), jnp.float32),
            pltpu.SemaphoreType.DMA((2,)),
            pltpu.SemaphoreType.DMA((_RING,)),
            pltpu.SemaphoreType.DMA((_RING,)),
        ],
        compiler_params=pltpu.CompilerParams(use_tc_tiling_on_sc=False),
    )
    def k(tab_hbm, src_hbm, dst_hbm, zero_hbm, out_hbm,
          sbufs, dbufs, rows, zbuf, acc, isem, gsem, ssem):
        ci = lax.axis_index("c")
        si = lax.axis_index("s")
        wid = ci * _NS + si
        rbase = wid * _NBATCH
        pltpu.sync_copy(zero_hbm, zbuf)

        def idx_copies(chunk, p):
            s_ = pltpu.make_async_copy(
                src_hbm.at[pl.ds(rbase + chunk * _CH, _CH)], sbufs.at[p],
                isem.at[p])
            d_ = pltpu.make_async_copy(
                dst_hbm.at[pl.ds(rbase + chunk * _CH, _CH)], dbufs.at[p],
                isem.at[p])
            return s_, d_

        for c in range(C):
            tab_c = tab_hbm.at[c]

            @pl.loop(0, _RPS // _ZB)
            def _(j):
                pltpu.sync_copy(zbuf, acc.at[pl.ds(si * _RPS + j * _ZB, _ZB)])

            plsc.subcore_barrier()

            s0, d0 = idx_copies(0, 0)
            s0.start()
            d0.start()

            @pl.loop(0, _NCHUNK, step=2)
            def _(kk):
                for pp in range(2):
                    chunk = kk + pp

                    @pl.when(chunk + 1 < _NCHUNK)
                    def _():
                        s_, d_ = idx_copies(chunk + 1, 1 - pp)
                        s_.start()
                        d_.start()

                    s_w, d_w = idx_copies(chunk, pp)
                    s_w.wait()
                    d_w.wait()
                    sbuf = sbufs.at[pp]
                    dbuf = dbufs.at[pp]

                    gd, sd = {}, {}
                    for j in range(_LOOK):
                        gd[j] = pltpu.async_copy(
                            tab_c.at[sbuf.at[j]], rows.at[j % _RING],
                            gsem.at[j % _RING])
                    for j in range(_CH):
                        b = j % _RING
                        gd[j].wait()
                        sd[j] = pltpu.async_copy(
                            rows.at[b], acc.at[dbuf.at[j]], ssem.at[b],
                            add=True)
                        jj = j + _LOOK
                        if jj < _CH:
                            if jj - _RING >= 0:
                                sd[jj - _RING].wait()
                            gd[jj] = pltpu.async_copy(
                                tab_c.at[sbuf.at[jj]], rows.at[jj % _RING],
                                gsem.at[jj % _RING])
                    for j in range(max(0, _CH - _RING), _CH):
                        sd[j].wait()

            plsc.subcore_barrier()
            pltpu.sync_copy(
                acc.at[pl.ds(si * _RPS, _RPS)],
                out_hbm.at[ci].at[pl.ds(si * _RPS, _RPS), pl.ds(c * _L, _L)])
            plsc.subcore_barrier()

    return k(tables, srcp2d, dstp2d, zeros)


def _lin1(p1, x, W1l, W1r, b1):
    def body(p_ref, x_ref, wl_ref, wr_ref, b_ref, y_ref, st_ref):
        p = p_ref[0] + p_ref[1]
        cnt = p[:, 8:9]
        inv = 1.0 / jnp.maximum(cnt, 1.0)
        mean8 = p[:, :8] * inv
        y = (lax.dot_general(mean8, wl_ref[...], (((1,), (1,)), ((), ())),
                             preferred_element_type=jnp.float32)
             + lax.dot_general(x_ref[...], wr_ref[...], (((1,), (1,)), ((), ())),
                               preferred_element_type=jnp.float32)
             + b_ref[...])
        y_ref[...] = y

        @pl.when(pl.program_id(0) == 0)
        def _():
            st_ref[...] = jnp.zeros_like(st_ref)

        st_ref[...] += jnp.stack([jnp.sum(y, axis=0), jnp.sum(y * y, axis=0)])

    return pl.pallas_call(
        body,
        grid=(_G,),
        in_specs=[
            pl.BlockSpec((2, _BLK, _L), lambda i: (0, i, 0)),
            pl.BlockSpec((_BLK, 8), lambda i: (i, 0)),
            pl.BlockSpec((64, 8), lambda i: (0, 0)),
            pl.BlockSpec((64, 8), lambda i: (0, 0)),
            pl.BlockSpec((64,), lambda i: (0,)),
        ],
        out_specs=[
            pl.BlockSpec((_BLK, 64), lambda i: (i, 0)),
            pl.BlockSpec((2, 64), lambda i: (0, 0)),
        ],
        out_shape=[
            jax.ShapeDtypeStruct((_N, 64), jnp.float32),
            jax.ShapeDtypeStruct((2, 64), jnp.float32),
        ],
    )(p1, x, W1l, W1r, b1)


def _bnrelu(y, st, g, be):
    def body(y_ref, st_ref, g_ref, be_ref, h_ref, hc_ref):
        yv = y_ref[...]
        mu = st_ref[0:1, :] / _N
        var = st_ref[1:2, :] / _N - mu * mu
        h = jnp.maximum((yv - mu) * lax.rsqrt(var + _EPS) * g_ref[...]
                        + be_ref[...], 0.0)
        h_ref[...] = h
        for c in range(4):
            hc_ref[c] = h[:, _L * c:_L * (c + 1)]

    return pl.pallas_call(
        body,
        grid=(_G,),
        in_specs=[
            pl.BlockSpec((_BLK, 64), lambda i: (i, 0)),
            pl.BlockSpec((2, 64), lambda i: (0, 0)),
            pl.BlockSpec((64,), lambda i: (0,)),
            pl.BlockSpec((64,), lambda i: (0,)),
        ],
        out_specs=[
            pl.BlockSpec((_BLK, 64), lambda i: (i, 0)),
            pl.BlockSpec((4, _BLK, _L), lambda i: (0, i, 0)),
        ],
        out_shape=[
            jax.ShapeDtypeStruct((_N, 64), jnp.float32),
            jax.ShapeDtypeStruct((4, _N, _L), jnp.float32),
        ],
    )(y, st, g, be)


def _lin2(p2, p1, h1, W2l, W2r, b2):
    def body(p2_ref, p1_ref, h_ref, wl_ref, wr_ref, b_ref, y_ref, st_ref):
        cnt = p1_ref[0, :, 8:9] + p1_ref[1, :, 8:9]
        inv = 1.0 / jnp.maximum(cnt, 1.0)
        agg = (p2_ref[0] + p2_ref[1]) * inv
        y = (lax.dot_general(agg, wl_ref[...], (((1,), (1,)), ((), ())),
                             preferred_element_type=jnp.float32)
             + lax.dot_general(h_ref[...], wr_ref[...], (((1,), (1,)), ((), ())),
                               preferred_element_type=jnp.float32)
             + b_ref[...])
        y_ref[...] = y

        @pl.when(pl.program_id(0) == 0)
        def _():
            st_ref[...] = jnp.zeros_like(st_ref)

        st_ref[...] += jnp.stack([jnp.sum(y, axis=0), jnp.sum(y * y, axis=0)])

    return pl.pallas_call(
        body,
        grid=(_G,),
        in_specs=[
            pl.BlockSpec((2, _BLK, 64), lambda i: (0, i, 0)),
            pl.BlockSpec((2, _BLK, _L), lambda i: (0, i, 0)),
            pl.BlockSpec((_BLK, 64), lambda i: (i, 0)),
            pl.BlockSpec((64, 64), lambda i: (0, 0)),
            pl.BlockSpec((64, 64), lambda i: (0, 0)),
            pl.BlockSpec((64,), lambda i: (0,)),
        ],
        out_specs=[
            pl.BlockSpec((_BLK, 64), lambda i: (i, 0)),
            pl.BlockSpec((2, 64), lambda i: (0, 0)),
        ],
        out_shape=[
            jax.ShapeDtypeStruct((_N, 64), jnp.float32),
            jax.ShapeDtypeStruct((2, 64), jnp.float32),
        ],
    )(p2, p1, h1, W2l, W2r, b2)


def _bnrelu_t3(y2, st2, g2, be2, W3l, W3r):
    def body(y_ref, st_ref, g_ref, be_ref, wl_ref, wr_ref, t_ref, r_ref):
        yv = y_ref[...]
        mu = st_ref[0:1, :] / _N
        var = st_ref[1:2, :] / _N - mu * mu
        h = jnp.maximum((yv - mu) * lax.rsqrt(var + _EPS) * g_ref[...]
                        + be_ref[...], 0.0)
        t3 = lax.dot_general(h, wl_ref[...], (((1,), (1,)), ((), ())),
                             preferred_element_type=jnp.float32)
        r3 = lax.dot_general(h, wr_ref[...], (((1,), (1,)), ((), ())),
                             preferred_element_type=jnp.float32)
        t_ref[0] = t3[:, :16]
        t_ref[1] = t3[:, 16:32]
        r_ref[...] = r3

    return pl.pallas_call(
        body,
        grid=(_G,),
        in_specs=[
            pl.BlockSpec((_BLK, 64), lambda i: (i, 0)),
            pl.BlockSpec((2, 64), lambda i: (0, 0)),
            pl.BlockSpec((64,), lambda i: (0,)),
            pl.BlockSpec((64,), lambda i: (0,)),
            pl.BlockSpec((32, 64), lambda i: (0, 0)),
            pl.BlockSpec((32, 64), lambda i: (0, 0)),
        ],
        out_specs=[
            pl.BlockSpec((2, _BLK, _L), lambda i: (0, i, 0)),
            pl.BlockSpec((_BLK, 32), lambda i: (i, 0)),
        ],
        out_shape=[
            jax.ShapeDtypeStruct((2, _N, _L), jnp.float32),
            jax.ShapeDtypeStruct((_N, 32), jnp.float32),
        ],
    )(y2, st2, g2, be2, W3l, W3r)


def _final(p3, p1, r3, b3, Wfc, bfc):
    def body(p3_ref, p1_ref, r_ref, b_ref, w_ref, bf_ref, o_ref):
        cnt = p1_ref[0, :, 8:9] + p1_ref[1, :, 8:9]
        inv = 1.0 / jnp.maximum(cnt, 1.0)
        agg = (p3_ref[0] + p3_ref[1]) * inv
        h3 = jnp.maximum(agg + r_ref[...] + b_ref[...], 0.0)
        logit = jnp.sum(h3 * w_ref[...], axis=1, keepdims=True) + bf_ref[0]
        o_ref[...] = 1.0 / (1.0 + jnp.exp(-logit))

    return pl.pallas_call(
        body,
        grid=(_G,),
        in_specs=[
            pl.BlockSpec((2, _BLK, 32), lambda i: (0, i, 0)),
            pl.BlockSpec((2, _BLK, _L), lambda i: (0, i, 0)),
            pl.BlockSpec((_BLK, 32), lambda i: (i, 0)),
            pl.BlockSpec((32,), lambda i: (0,)),
            pl.BlockSpec((1, 32), lambda i: (0, 0)),
            pl.BlockSpec((1,), lambda i: (0,)),
        ],
        out_specs=pl.BlockSpec((_BLK, 1), lambda i: (i, 0)),
        out_shape=jax.ShapeDtypeStruct((_N, 1), jnp.float32),
    )(p3, p1, r3, b3, Wfc, bfc)


def kernel(x, edge_index, W1l, W1r, b1, g1, be1, W2l, W2r, b2, g2, be2,
           W3l, W3r, b3, Wfc, bfc):
    src = edge_index[0]
    dst = edge_index[1]
    pad = _EP - _E
    srcp = jnp.concatenate([src, jnp.zeros((pad,), jnp.int32)]).reshape(-1, _B)
    dstp = jnp.concatenate([dst, jnp.full((pad,), _N, jnp.int32)]).reshape(-1, _B)
    zeros = jnp.zeros((_ZB, _L), jnp.float32)
    xaug = jnp.concatenate(
        [x, jnp.ones((_N, 1), jnp.float32), jnp.zeros((_N, 7), jnp.float32)],
        axis=1)[None]

    p1 = _segsum(xaug, srcp, dstp, zeros)
    y1, st1 = _lin1(p1, x, W1l, W1r, b1)
    h1, h1c = _bnrelu(y1, st1, g1, be1)
    p2 = _segsum(h1c, srcp, dstp, zeros)
    y2, st2 = _lin2(p2, p1, h1, W2l, W2r, b2)
    t3c, r3 = _bnrelu_t3(y2, st2, g2, be2, W3l, W3r)
    p3 = _segsum(t3c, srcp, dstp, zeros)
    o = _final(p3, p1, r3, b3, Wfc, bfc)
    return jnp.squeeze(o, axis=-1)
